# Initial kernel scaffold; baseline (speedup 1.0000x reference)
import jax
import jax.numpy as jnp
from jax.experimental import pallas as pl


def kernel(x_wide, x_deep, emb, W_wide, b_wide, W1, b1, W2, b2, W3, b3):
    raise NotImplementedError("write your pallas kernel here")


# speedup vs baseline: 7.7166x; 7.7166x over previous
import functools

import jax
import jax.numpy as jnp
from jax import lax
from jax.experimental import pallas as pl
from jax.experimental.pallas import tpu as pltpu
from jax.experimental.pallas import tpu_sc as plsc

B = 16384
DENSE = 13
FIELDS = 26
VOCAB = 100000
K = 16
H1, H2 = 256, 128

R = B * FIELDS
NC, NS = 2, 16
NW = NC * NS
R_PER_W = R // NW
CH = 1664
NCHUNK = R_PER_W // CH

BLK = 1024


def _gather_sc(flat_idx, table):
  mesh = plsc.VectorSubcoreMesh(core_axis_name="c", subcore_axis_name="s")

  @functools.partial(
      pl.kernel,
      mesh=mesh,
      out_type=jax.ShapeDtypeStruct((R, K), jnp.float32),
      scratch_types=[
          pltpu.VMEM((CH,), jnp.int32),
          pltpu.VMEM((CH, K), jnp.float32),
          pltpu.SemaphoreType.DMA,
      ],
      compiler_params=pltpu.CompilerParams(use_tc_tiling_on_sc=False),
  )
  def gather_kernel(idx_hbm, table_hbm, out_hbm, idx_v, rows_v, sem):
    wid = lax.axis_index("s") * NC + lax.axis_index("c")
    base = wid * R_PER_W
    for c in range(NCHUNK):
      off = base + c * CH
      pltpu.sync_copy(idx_hbm.at[pl.ds(off, CH)], idx_v)
      pltpu.async_copy(table_hbm.at[idx_v], rows_v, sem).wait()
      pltpu.sync_copy(rows_v, out_hbm.at[pl.ds(off, CH)])

  return gather_kernel(flat_idx, table)


def _mlp_body(xw_ref, sp_ref, w1a, w1b, b1r, w2, b2r, ww, bwr, w3w, w3d, b3r,
              out_ref):
  xw = xw_ref[...]
  sp = sp_ref[...]
  h = jnp.dot(sp, w1b[...], preferred_element_type=jnp.float32)
  h = h + jnp.dot(xw, w1a[...], preferred_element_type=jnp.float32) + b1r[...]
  h = jnp.where(h >= 0, h, 0.01 * h)
  d = jnp.dot(h, w2[...], preferred_element_type=jnp.float32) + b2r[...]
  d = jnp.where(d >= 0, d, 0.01 * d)
  wide = jnp.dot(xw, ww[...], preferred_element_type=jnp.float32) + bwr[...]
  z = wide * w3w[...] + jnp.dot(d, w3d[...],
                                preferred_element_type=jnp.float32) + b3r[...]
  out_ref[...] = 1.0 / (1.0 + jnp.exp(-z))


def _mlp_tc(xw_p, sp, w1a, w1b, b1r, w2, b2r, ww, bwr, w3w, w3d, b3r):
  n = B // BLK
  full = lambda shape: pl.BlockSpec(shape, lambda i: (0, 0))
  return pl.pallas_call(
      _mlp_body,
      grid=(n,),
      in_specs=[
          pl.BlockSpec((BLK, 16), lambda i: (i, 0)),
          pl.BlockSpec((BLK, FIELDS * K), lambda i: (i, 0)),
          full((16, H1)),
          full((FIELDS * K, H1)),
          full((1, H1)),
          full((H1, H2)),
          full((1, H2)),
          full((16, 1)),
          full((1, 1)),
          full((1, 1)),
          full((H2, 1)),
          full((1, 1)),
      ],
      out_specs=pl.BlockSpec((BLK, 1), lambda i: (i, 0)),
      out_shape=jax.ShapeDtypeStruct((B, 1), jnp.float32),
      compiler_params=pltpu.CompilerParams(
          dimension_semantics=("parallel",)),
  )(xw_p, sp, w1a, w1b, b1r, w2, b2r, ww, bwr, w3w, w3d, b3r)


def kernel(x_wide, x_deep, emb, W_wide, b_wide, W1, b1, W2, b2, W3, b3):
  offs = (jnp.arange(FIELDS, dtype=jnp.int32) * VOCAB)[None, :]
  flat_idx = (x_deep.astype(jnp.int32) + offs).reshape(R)
  table = emb.reshape(FIELDS * VOCAB, K)
  rows = _gather_sc(flat_idx, table)
  sp = rows.reshape(B, FIELDS * K)

  zeros3 = jnp.zeros((3,) + W1.shape[1:], jnp.float32)
  xw_p = jnp.concatenate([x_wide, jnp.zeros((B, 3), jnp.float32)], axis=1)
  w1a = jnp.concatenate([W1[:DENSE], zeros3], axis=0)
  w1b = W1[DENSE:]
  ww = jnp.concatenate([W_wide, jnp.zeros((3, 1), jnp.float32)], axis=0)
  return _mlp_tc(
      xw_p, sp, w1a, w1b, b1.reshape(1, H1), W2, b2.reshape(1, H2),
      ww, b_wide.reshape(1, 1), W3[:1], W3[1:], b3.reshape(1, 1))


# speedup vs baseline: 7.7544x; 1.0049x over previous
import functools

import jax
import jax.numpy as jnp
from jax import lax
from jax.experimental import pallas as pl
from jax.experimental.pallas import tpu as pltpu
from jax.experimental.pallas import tpu_sc as plsc

B = 16384
DENSE = 13
FIELDS = 26
VOCAB = 100000
K = 16
H1, H2 = 256, 128

R = B * FIELDS
NC, NS = 2, 16
NW = NC * NS
CHUNK = 128
CH_W = R // NW // CHUNK
GRP = 8
NGRP = CH_W // GRP

BLK = 1024


def _gather_sc(idx2d, table):
  mesh = plsc.VectorSubcoreMesh(core_axis_name="c", subcore_axis_name="s")

  @functools.partial(
      pl.kernel,
      mesh=mesh,
      out_type=jax.ShapeDtypeStruct((R, K), jnp.float32),
      scratch_types=[
          pltpu.VMEM((CH_W, CHUNK), jnp.int32),
          pltpu.VMEM((GRP, CHUNK, K), jnp.float32),
          pltpu.SemaphoreType.DMA,
          pltpu.SemaphoreType.DMA,
      ],
      compiler_params=pltpu.CompilerParams(use_tc_tiling_on_sc=False),
  )
  def gather_kernel(idx_hbm, tab_hbm, out_hbm, idx_v, rows_v, sem_g, sem_o):
    wid = lax.axis_index("s") * NC + lax.axis_index("c")
    c0 = wid * CH_W
    pltpu.sync_copy(idx_hbm.at[pl.ds(c0, CH_W)], idx_v)

    @pl.loop(0, NGRP)
    def _group(g):
      j0 = g * GRP
      g_cps = [
          pltpu.async_copy(tab_hbm.at[idx_v.at[j0 + b]], rows_v.at[b], sem_g)
          for b in range(GRP)
      ]
      o_cps = []
      for b in range(GRP):
        g_cps[b].wait()
        o_cps.append(
            pltpu.async_copy(rows_v.at[b],
                             out_hbm.at[pl.ds((c0 + j0 + b) * CHUNK, CHUNK)],
                             sem_o))
      for cp in o_cps:
        cp.wait()

  return gather_kernel(idx2d, table)


def _mlp_body(xw_ref, sp_ref, w1a, w1b, b1r, w2, b2r, ww, bwr, w3w, w3d, b3r,
              out_ref):
  xw = xw_ref[...]
  sp = sp_ref[...]
  h = jnp.dot(sp, w1b[...], preferred_element_type=jnp.float32)
  h = h + jnp.dot(xw, w1a[...], preferred_element_type=jnp.float32) + b1r[...]
  h = jnp.where(h >= 0, h, 0.01 * h)
  d = jnp.dot(h, w2[...], preferred_element_type=jnp.float32) + b2r[...]
  d = jnp.where(d >= 0, d, 0.01 * d)
  wide = jnp.dot(xw, ww[...], preferred_element_type=jnp.float32) + bwr[...]
  z = wide * w3w[...] + jnp.dot(d, w3d[...],
                                preferred_element_type=jnp.float32) + b3r[...]
  out_ref[...] = 1.0 / (1.0 + jnp.exp(-z))


def _mlp_tc(xw_p, sp, w1a, w1b, b1r, w2, b2r, ww, bwr, w3w, w3d, b3r):
  n = B // BLK
  full = lambda shape: pl.BlockSpec(shape, lambda i: (0, 0))
  return pl.pallas_call(
      _mlp_body,
      grid=(n,),
      in_specs=[
          pl.BlockSpec((BLK, 16), lambda i: (i, 0)),
          pl.BlockSpec((BLK, FIELDS * K), lambda i: (i, 0)),
          full((16, H1)),
          full((FIELDS * K, H1)),
          full((1, H1)),
          full((H1, H2)),
          full((1, H2)),
          full((16, 1)),
          full((1, 1)),
          full((1, 1)),
          full((H2, 1)),
          full((1, 1)),
      ],
      out_specs=pl.BlockSpec((BLK, 1), lambda i: (i, 0)),
      out_shape=jax.ShapeDtypeStruct((B, 1), jnp.float32),
      compiler_params=pltpu.CompilerParams(
          dimension_semantics=("parallel",)),
  )(xw_p, sp, w1a, w1b, b1r, w2, b2r, ww, bwr, w3w, w3d, b3r)


def kernel(x_wide, x_deep, emb, W_wide, b_wide, W1, b1, W2, b2, W3, b3):
  offs = (jnp.arange(FIELDS, dtype=jnp.int32) * VOCAB)[None, :]
  idx2d = (x_deep.astype(jnp.int32) + offs).reshape(R // CHUNK, CHUNK)
  table = emb.reshape(FIELDS * VOCAB, K)
  rows = _gather_sc(idx2d, table)
  sp = rows.reshape(B, FIELDS * K)

  zeros3 = jnp.zeros((3,) + W1.shape[1:], jnp.float32)
  xw_p = jnp.concatenate([x_wide, jnp.zeros((B, 3), jnp.float32)], axis=1)
  w1a = jnp.concatenate([W1[:DENSE], zeros3], axis=0)
  w1b = W1[DENSE:]
  ww = jnp.concatenate([W_wide, jnp.zeros((3, 1), jnp.float32)], axis=0)
  return _mlp_tc(
      xw_p, sp, w1a, w1b, b1.reshape(1, H1), W2, b2.reshape(1, H2),
      ww, b_wide.reshape(1, 1), W3[:1], W3[1:], b3.reshape(1, 1))


# speedup vs baseline: 8.2563x; 1.0647x over previous
import functools

import jax
import jax.numpy as jnp
from jax import lax
from jax.experimental import pallas as pl
from jax.experimental.pallas import tpu as pltpu
from jax.experimental.pallas import tpu_sc as plsc

B = 16384
DENSE = 13
FIELDS = 26
VOCAB = 100000
K = 16
H1, H2 = 256, 128

R = B * FIELDS
NC, NS = 2, 16
NW = NC * NS
CHUNK = 128
CH_W = R // NW // CHUNK
GRP = 8
NGRP = CH_W // GRP

BLK = 1024


def _gather_sc(idx2d, table):
  mesh = plsc.VectorSubcoreMesh(core_axis_name="c", subcore_axis_name="s")

  @functools.partial(
      pl.kernel,
      mesh=mesh,
      out_type=jax.ShapeDtypeStruct((R, K), jnp.float32),
      scratch_types=[
          pltpu.VMEM((CH_W, CHUNK), jnp.int32),
          pltpu.VMEM((GRP, CHUNK, K), jnp.float32),
          pltpu.SemaphoreType.DMA,
          pltpu.SemaphoreType.DMA,
      ],
      compiler_params=pltpu.CompilerParams(use_tc_tiling_on_sc=False),
  )
  def gather_kernel(idx_hbm, tab_hbm, out_hbm, idx_v, rows_v, sem_g, sem_o):
    wid = lax.axis_index("s") * NC + lax.axis_index("c")
    c0 = wid * CH_W
    pltpu.sync_copy(idx_hbm.at[pl.ds(c0, CH_W)], idx_v)

    @pl.loop(0, NGRP)
    def _group(g):
      j0 = g * GRP
      g_cps = [
          pltpu.async_copy(tab_hbm.at[idx_v.at[j0 + b]], rows_v.at[b], sem_g)
          for b in range(GRP)
      ]
      o_cps = []
      for b in range(GRP):
        g_cps[b].wait()
        o_cps.append(
            pltpu.async_copy(rows_v.at[b],
                             out_hbm.at[pl.ds((c0 + j0 + b) * CHUNK, CHUNK)],
                             sem_o))
      for cp in o_cps:
        cp.wait()

  return gather_kernel(idx2d, table)


DT_RB = 8000


def _detile_body(x_ref, o_ref):
  x3 = x_ref[...].reshape(DT_RB // 8, 8, K)
  o_ref[...] = jnp.concatenate([x3[:, j, :] for j in range(8)], axis=1)


def _detile_tc(table2d):
  return pl.pallas_call(
      _detile_body,
      grid=(FIELDS * VOCAB // DT_RB,),
      in_specs=[pl.BlockSpec((DT_RB, K), lambda i: (i, 0))],
      out_specs=pl.BlockSpec((DT_RB // 8, 128), lambda i: (i, 0)),
      out_shape=jax.ShapeDtypeStruct((FIELDS * VOCAB // 8, 128), jnp.float32),
      compiler_params=pltpu.CompilerParams(
          dimension_semantics=("arbitrary",)),
  )(table2d)


def _mlp_body(xw_ref, sp_ref, w1a, w1b, b1r, w2, b2r, ww, bwr, w3w, w3d, b3r,
              out_ref):
  xw = xw_ref[...]
  sp = sp_ref[...]
  h = jnp.dot(sp, w1b[...], preferred_element_type=jnp.float32)
  h = h + jnp.dot(xw, w1a[...], preferred_element_type=jnp.float32) + b1r[...]
  h = jnp.where(h >= 0, h, 0.01 * h)
  d = jnp.dot(h, w2[...], preferred_element_type=jnp.float32) + b2r[...]
  d = jnp.where(d >= 0, d, 0.01 * d)
  wide = jnp.dot(xw, ww[...], preferred_element_type=jnp.float32) + bwr[...]
  z = wide * w3w[...] + jnp.dot(d, w3d[...],
                                preferred_element_type=jnp.float32) + b3r[...]
  out_ref[...] = 1.0 / (1.0 + jnp.exp(-z))


def _mlp_tc(xw_p, sp, w1a, w1b, b1r, w2, b2r, ww, bwr, w3w, w3d, b3r):
  n = B // BLK
  full = lambda shape: pl.BlockSpec(shape, lambda i: (0, 0))
  return pl.pallas_call(
      _mlp_body,
      grid=(n,),
      in_specs=[
          pl.BlockSpec((BLK, 16), lambda i: (i, 0)),
          pl.BlockSpec((BLK, FIELDS * K), lambda i: (i, 0)),
          full((16, H1)),
          full((FIELDS * K, H1)),
          full((1, H1)),
          full((H1, H2)),
          full((1, H2)),
          full((16, 1)),
          full((1, 1)),
          full((1, 1)),
          full((H2, 1)),
          full((1, 1)),
      ],
      out_specs=pl.BlockSpec((BLK, 1), lambda i: (i, 0)),
      out_shape=jax.ShapeDtypeStruct((B, 1), jnp.float32),
      compiler_params=pltpu.CompilerParams(
          dimension_semantics=("parallel",)),
  )(xw_p, sp, w1a, w1b, b1r, w2, b2r, ww, bwr, w3w, w3d, b3r)


def kernel(x_wide, x_deep, emb, W_wide, b_wide, W1, b1, W2, b2, W3, b3):
  offs = (jnp.arange(FIELDS, dtype=jnp.int32) * VOCAB)[None, :]
  idx2d = (x_deep.astype(jnp.int32) + offs).reshape(R // CHUNK, CHUNK)
  table = _detile_tc(emb.reshape(FIELDS * VOCAB, K)).reshape(FIELDS * VOCAB, K)
  rows = _gather_sc(idx2d, table)
  sp = rows.reshape(B, FIELDS * K)

  zeros3 = jnp.zeros((3,) + W1.shape[1:], jnp.float32)
  xw_p = jnp.concatenate([x_wide, jnp.zeros((B, 3), jnp.float32)], axis=1)
  w1a = jnp.concatenate([W1[:DENSE], zeros3], axis=0)
  w1b = W1[DENSE:]
  ww = jnp.concatenate([W_wide, jnp.zeros((3, 1), jnp.float32)], axis=0)
  return _mlp_tc(
      xw_p, sp, w1a, w1b, b1.reshape(1, H1), W2, b2.reshape(1, H2),
      ww, b_wide.reshape(1, 1), W3[:1], W3[1:], b3.reshape(1, 1))


# speedup vs baseline: 8.2589x; 1.0003x over previous
import functools

import jax
import jax.numpy as jnp
from jax import lax
from jax.experimental import pallas as pl
from jax.experimental.pallas import tpu as pltpu
from jax.experimental.pallas import tpu_sc as plsc

B = 16384
DENSE = 13
FIELDS = 26
VOCAB = 100000
K = 16
H1, H2 = 256, 128

R = B * FIELDS
NC, NS = 2, 16
NW = NC * NS
CHUNK = 128
CH_W = R // NW // CHUNK
GRP = 8
NGRP = CH_W // GRP

BLK = 1024


def _gather_sc(idx2d, table):
  mesh = plsc.VectorSubcoreMesh(core_axis_name="c", subcore_axis_name="s")

  @functools.partial(
      pl.kernel,
      mesh=mesh,
      out_type=jax.ShapeDtypeStruct((R, K), jnp.float32),
      scratch_types=[
          pltpu.VMEM((CH_W, CHUNK), jnp.int32),
          pltpu.VMEM((GRP, CHUNK, K), jnp.float32),
          pltpu.SemaphoreType.DMA,
          pltpu.SemaphoreType.DMA,
      ],
      compiler_params=pltpu.CompilerParams(use_tc_tiling_on_sc=False),
  )
  def gather_kernel(idx_hbm, tab_hbm, out_hbm, idx_v, rows_v, sem_g, sem_o):
    wid = lax.axis_index("s") * NC + lax.axis_index("c")
    c0 = wid * CH_W
    pltpu.sync_copy(idx_hbm.at[pl.ds(c0, CH_W)], idx_v)

    @pl.loop(0, NGRP)
    def _group(g):
      j0 = g * GRP
      g_cps = [
          pltpu.async_copy(tab_hbm.at[idx_v.at[j0 + b]], rows_v.at[b], sem_g)
          for b in range(GRP)
      ]
      o_cps = []
      for b in range(GRP):
        g_cps[b].wait()
        o_cps.append(
            pltpu.async_copy(rows_v.at[b],
                             out_hbm.at[pl.ds((c0 + j0 + b) * CHUNK, CHUNK)],
                             sem_o))
      for cp in o_cps:
        cp.wait()

  return gather_kernel(idx2d, table)


DT_RB = 8000


def _detile_body(x_ref, e_ref, o_ref):
  x3 = x_ref[...].reshape(DT_RB // 8, 8, K)
  acc = jnp.dot(x3[:, 0, :], e_ref[0], preferred_element_type=jnp.float32)
  for j in range(1, 8):
    acc += jnp.dot(x3[:, j, :], e_ref[j], preferred_element_type=jnp.float32)
  o_ref[...] = acc


def _detile_tc(table2d, esel):
  return pl.pallas_call(
      _detile_body,
      grid=(FIELDS * VOCAB // DT_RB,),
      in_specs=[
          pl.BlockSpec((DT_RB, K), lambda i: (i, 0)),
          pl.BlockSpec((8, K, 128), lambda i: (0, 0, 0)),
      ],
      out_specs=pl.BlockSpec((DT_RB // 8, 128), lambda i: (i, 0)),
      out_shape=jax.ShapeDtypeStruct((FIELDS * VOCAB // 8, 128), jnp.float32),
      compiler_params=pltpu.CompilerParams(
          dimension_semantics=("arbitrary",)),
  )(table2d, esel)


def _mlp_body(xw_ref, sp_ref, w1a, w1b, b1r, w2, b2r, ww, bwr, w3w, w3d, b3r,
              out_ref):
  xw = xw_ref[...]
  sp = sp_ref[...]
  h = jnp.dot(sp, w1b[...], preferred_element_type=jnp.float32)
  h = h + jnp.dot(xw, w1a[...], preferred_element_type=jnp.float32) + b1r[...]
  h = jnp.where(h >= 0, h, 0.01 * h)
  d = jnp.dot(h, w2[...], preferred_element_type=jnp.float32) + b2r[...]
  d = jnp.where(d >= 0, d, 0.01 * d)
  wide = jnp.dot(xw, ww[...], preferred_element_type=jnp.float32) + bwr[...]
  z = wide * w3w[...] + jnp.dot(d, w3d[...],
                                preferred_element_type=jnp.float32) + b3r[...]
  out_ref[...] = 1.0 / (1.0 + jnp.exp(-z))


def _mlp_tc(xw_p, sp, w1a, w1b, b1r, w2, b2r, ww, bwr, w3w, w3d, b3r):
  n = B // BLK
  full = lambda shape: pl.BlockSpec(shape, lambda i: (0, 0))
  return pl.pallas_call(
      _mlp_body,
      grid=(n,),
      in_specs=[
          pl.BlockSpec((BLK, 16), lambda i: (i, 0)),
          pl.BlockSpec((BLK, FIELDS * K), lambda i: (i, 0)),
          full((16, H1)),
          full((FIELDS * K, H1)),
          full((1, H1)),
          full((H1, H2)),
          full((1, H2)),
          full((16, 1)),
          full((1, 1)),
          full((1, 1)),
          full((H2, 1)),
          full((1, 1)),
      ],
      out_specs=pl.BlockSpec((BLK, 1), lambda i: (i, 0)),
      out_shape=jax.ShapeDtypeStruct((B, 1), jnp.float32),
      compiler_params=pltpu.CompilerParams(
          dimension_semantics=("parallel",)),
  )(xw_p, sp, w1a, w1b, b1r, w2, b2r, ww, bwr, w3w, w3d, b3r)


def kernel(x_wide, x_deep, emb, W_wide, b_wide, W1, b1, W2, b2, W3, b3):
  offs = (jnp.arange(FIELDS, dtype=jnp.int32) * VOCAB)[None, :]
  idx2d = (x_deep.astype(jnp.int32) + offs).reshape(R // CHUNK, CHUNK)
  esel = jnp.stack(
      [jnp.eye(K, 128, k=j * K, dtype=jnp.float32) for j in range(8)])
  table = _detile_tc(emb.reshape(FIELDS * VOCAB, K),
                     esel).reshape(FIELDS * VOCAB, K)
  rows = _gather_sc(idx2d, table)
  sp = rows.reshape(B, FIELDS * K)

  zeros3 = jnp.zeros((3,) + W1.shape[1:], jnp.float32)
  xw_p = jnp.concatenate([x_wide, jnp.zeros((B, 3), jnp.float32)], axis=1)
  w1a = jnp.concatenate([W1[:DENSE], zeros3], axis=0)
  w1b = W1[DENSE:]
  ww = jnp.concatenate([W_wide, jnp.zeros((3, 1), jnp.float32)], axis=0)
  return _mlp_tc(
      xw_p, sp, w1a, w1b, b1.reshape(1, H1), W2, b2.reshape(1, H2),
      ww, b_wide.reshape(1, 1), W3[:1], W3[1:], b3.reshape(1, 1))


# speedup vs baseline: 10.7606x; 1.3029x over previous
import functools

import jax
import jax.numpy as jnp
from jax import lax
from jax.experimental import pallas as pl
from jax.experimental.pallas import tpu as pltpu
from jax.experimental.pallas import tpu_sc as plsc

B = 16384
DENSE = 13
FIELDS = 26
VOCAB = 100000
K = 16
H1, H2 = 256, 128

R = B * FIELDS
NC, NS = 2, 16
NW = NC * NS
CHUNK = 128
CH_W = R // NW // CHUNK
GRP = 8
NGRP = CH_W // GRP

BLK = 1024


def _gather_sc(idx2d, table):
  mesh = plsc.VectorSubcoreMesh(core_axis_name="c", subcore_axis_name="s")

  @functools.partial(
      pl.kernel,
      mesh=mesh,
      out_type=jax.ShapeDtypeStruct((R, K), jnp.float32),
      scratch_types=[
          pltpu.VMEM((CH_W, CHUNK), jnp.int32),
          pltpu.VMEM((GRP, CHUNK, K), jnp.float32),
          pltpu.SemaphoreType.DMA,
          pltpu.SemaphoreType.DMA,
      ],
      compiler_params=pltpu.CompilerParams(use_tc_tiling_on_sc=False),
  )
  def gather_kernel(idx_hbm, tab_hbm, out_hbm, idx_v, rows_v, sem_g, sem_o):
    wid = lax.axis_index("s") * NC + lax.axis_index("c")
    c0 = wid * CH_W
    pltpu.sync_copy(idx_hbm.at[pl.ds(c0, CH_W)], idx_v)

    @pl.loop(0, NGRP)
    def _group(g):
      j0 = g * GRP
      g_cps = [
          pltpu.async_copy(tab_hbm.at[idx_v.at[j0 + b]], rows_v.at[b], sem_g)
          for b in range(GRP)
      ]
      o_cps = []
      for b in range(GRP):
        g_cps[b].wait()
        o_cps.append(
            pltpu.async_copy(rows_v.at[b],
                             out_hbm.at[pl.ds((c0 + j0 + b) * CHUNK, CHUNK)],
                             sem_o))
      for cp in o_cps:
        cp.wait()

  return gather_kernel(idx2d, table)


GPF = 12504
VOCAB_PAD = GPF * 8
DCH = 4096


def _detile_body(x_ref, e_ref, o_ref):
  col = 0
  while col < VOCAB:
    w = min(DCH, VOCAB - col)
    xt = x_ref[:, pl.ds(col, w)].T
    x3 = xt.reshape(w // 8, 8, K)
    acc = jnp.dot(x3[:, 0, :], e_ref[0], preferred_element_type=jnp.float32)
    for j in range(1, 8):
      acc += jnp.dot(x3[:, j, :], e_ref[j],
                     preferred_element_type=jnp.float32)
    o_ref[0, pl.ds(col // 8, w // 8), :] = acc
    col += w


def _detile_tc(tableT, esel):
  return pl.pallas_call(
      _detile_body,
      grid=(FIELDS,),
      in_specs=[
          pl.BlockSpec((K, VOCAB), lambda f: (f, 0)),
          pl.BlockSpec((8, K, 128), lambda f: (0, 0, 0)),
      ],
      out_specs=pl.BlockSpec((1, GPF, 128), lambda f: (f, 0, 0)),
      out_shape=jax.ShapeDtypeStruct((FIELDS, GPF, 128), jnp.float32),
      compiler_params=pltpu.CompilerParams(
          dimension_semantics=("arbitrary",)),
  )(tableT, esel)


def _mlp_body(xw_ref, sp_ref, w1a, w1b, b1r, w2, b2r, ww, bwr, w3w, w3d, b3r,
              out_ref):
  xw = xw_ref[...]
  sp = sp_ref[...]
  h = jnp.dot(sp, w1b[...], preferred_element_type=jnp.float32)
  h = h + jnp.dot(xw, w1a[...], preferred_element_type=jnp.float32) + b1r[...]
  h = jnp.where(h >= 0, h, 0.01 * h)
  d = jnp.dot(h, w2[...], preferred_element_type=jnp.float32) + b2r[...]
  d = jnp.where(d >= 0, d, 0.01 * d)
  wide = jnp.dot(xw, ww[...], preferred_element_type=jnp.float32) + bwr[...]
  z = wide * w3w[...] + jnp.dot(d, w3d[...],
                                preferred_element_type=jnp.float32) + b3r[...]
  out_ref[...] = 1.0 / (1.0 + jnp.exp(-z))


def _mlp_tc(xw_p, sp, w1a, w1b, b1r, w2, b2r, ww, bwr, w3w, w3d, b3r):
  n = B // BLK
  full = lambda shape: pl.BlockSpec(shape, lambda i: (0, 0))
  return pl.pallas_call(
      _mlp_body,
      grid=(n,),
      in_specs=[
          pl.BlockSpec((BLK, 16), lambda i: (i, 0)),
          pl.BlockSpec((BLK, FIELDS * K), lambda i: (i, 0)),
          full((16, H1)),
          full((FIELDS * K, H1)),
          full((1, H1)),
          full((H1, H2)),
          full((1, H2)),
          full((16, 1)),
          full((1, 1)),
          full((1, 1)),
          full((H2, 1)),
          full((1, 1)),
      ],
      out_specs=pl.BlockSpec((BLK, 1), lambda i: (i, 0)),
      out_shape=jax.ShapeDtypeStruct((B, 1), jnp.float32),
      compiler_params=pltpu.CompilerParams(
          dimension_semantics=("parallel",)),
  )(xw_p, sp, w1a, w1b, b1r, w2, b2r, ww, bwr, w3w, w3d, b3r)


def kernel(x_wide, x_deep, emb, W_wide, b_wide, W1, b1, W2, b2, W3, b3):
  offs = (jnp.arange(FIELDS, dtype=jnp.int32) * VOCAB_PAD)[None, :]
  idx2d = (x_deep.astype(jnp.int32) + offs).reshape(R // CHUNK, CHUNK)
  esel = jnp.stack(
      [jnp.eye(K, 128, k=j * K, dtype=jnp.float32) for j in range(8)])
  embT = jnp.transpose(emb, (0, 2, 1)).reshape(FIELDS * K, VOCAB)
  table = _detile_tc(embT, esel).reshape(FIELDS * VOCAB_PAD, K)
  rows = _gather_sc(idx2d, table)
  sp = rows.reshape(B, FIELDS * K)

  zeros3 = jnp.zeros((3,) + W1.shape[1:], jnp.float32)
  xw_p = jnp.concatenate([x_wide, jnp.zeros((B, 3), jnp.float32)], axis=1)
  w1a = jnp.concatenate([W1[:DENSE], zeros3], axis=0)
  w1b = W1[DENSE:]
  ww = jnp.concatenate([W_wide, jnp.zeros((3, 1), jnp.float32)], axis=0)
  return _mlp_tc(
      xw_p, sp, w1a, w1b, b1.reshape(1, H1), W2, b2.reshape(1, H2),
      ww, b_wide.reshape(1, 1), W3[:1], W3[1:], b3.reshape(1, 1))


# speedup vs baseline: 11.5935x; 1.0774x over previous
import functools

import jax
import jax.numpy as jnp
from jax import lax
from jax.experimental import pallas as pl
from jax.experimental.pallas import tpu as pltpu
from jax.experimental.pallas import tpu_sc as plsc

B = 16384
DENSE = 13
FIELDS = 26
VOCAB = 100000
K = 16
H1, H2 = 256, 128

R = B * FIELDS
NC, NS = 2, 16
NW = NC * NS
CHUNK = 128
CH_W = R // NW // CHUNK
GRP = 8
NGRP = CH_W // GRP

BLK = 1024


def _gather_sc(idx2d, table):
  mesh = plsc.VectorSubcoreMesh(core_axis_name="c", subcore_axis_name="s")

  @functools.partial(
      pl.kernel,
      mesh=mesh,
      out_type=jax.ShapeDtypeStruct((R, K), jnp.float32),
      scratch_types=[
          pltpu.VMEM((CH_W, CHUNK), jnp.int32),
          pltpu.VMEM((GRP, CHUNK, K), jnp.float32),
          pltpu.SemaphoreType.DMA,
          pltpu.SemaphoreType.DMA,
      ],
      compiler_params=pltpu.CompilerParams(use_tc_tiling_on_sc=False),
  )
  def gather_kernel(idx_hbm, tab_hbm, out_hbm, idx_v, rows_v, sem_g, sem_o):
    wid = lax.axis_index("s") * NC + lax.axis_index("c")
    c0 = wid * CH_W
    pltpu.sync_copy(idx_hbm.at[pl.ds(c0, CH_W)], idx_v)

    @pl.loop(0, NGRP)
    def _group(g):
      j0 = g * GRP
      g_cps = [
          pltpu.async_copy(tab_hbm.at[idx_v.at[j0 + b]], rows_v.at[b], sem_g)
          for b in range(GRP)
      ]
      o_cps = []
      for b in range(GRP):
        g_cps[b].wait()
        o_cps.append(
            pltpu.async_copy(rows_v.at[b],
                             out_hbm.at[pl.ds((c0 + j0 + b) * CHUNK, CHUNK)],
                             sem_o))
      for cp in o_cps:
        cp.wait()

  return gather_kernel(idx2d, table)


GPF = 12504
VOCAB_PAD = GPF * 8
DCH = 4096


def _detile_body(x_ref, e_ref, o_ref):
  col = 0
  while col < VOCAB:
    w = min(DCH, VOCAB - col)
    xt = lax.dot_general(
        x_ref[:, pl.ds(col, w)], jnp.eye(K, dtype=jnp.float32),
        (((0,), (0,)), ((), ())), preferred_element_type=jnp.float32)
    x3 = xt.reshape(w // 8, 8, K)
    acc = jnp.dot(x3[:, 0, :], e_ref[0], preferred_element_type=jnp.float32)
    for j in range(1, 8):
      acc += jnp.dot(x3[:, j, :], e_ref[j],
                     preferred_element_type=jnp.float32)
    o_ref[0, pl.ds(col // 8, w // 8), :] = acc
    col += w


def _detile_tc(tableT, esel):
  return pl.pallas_call(
      _detile_body,
      grid=(FIELDS,),
      in_specs=[
          pl.BlockSpec((K, VOCAB), lambda f: (f, 0)),
          pl.BlockSpec((8, K, 128), lambda f: (0, 0, 0)),
      ],
      out_specs=pl.BlockSpec((1, GPF, 128), lambda f: (f, 0, 0)),
      out_shape=jax.ShapeDtypeStruct((FIELDS, GPF, 128), jnp.float32),
      compiler_params=pltpu.CompilerParams(
          dimension_semantics=("arbitrary",)),
  )(tableT, esel)


def _mlp_body(xw_ref, sp_ref, w1a, w1b, b1r, w2, b2r, ww, bwr, w3w, w3d, b3r,
              out_ref):
  xw = xw_ref[...]
  sp = sp_ref[...]
  h = jnp.dot(sp, w1b[...], preferred_element_type=jnp.float32)
  h = h + jnp.dot(xw, w1a[...], preferred_element_type=jnp.float32) + b1r[...]
  h = jnp.where(h >= 0, h, 0.01 * h)
  d = jnp.dot(h, w2[...], preferred_element_type=jnp.float32) + b2r[...]
  d = jnp.where(d >= 0, d, 0.01 * d)
  wide = jnp.dot(xw, ww[...], preferred_element_type=jnp.float32) + bwr[...]
  z = wide * w3w[...] + jnp.dot(d, w3d[...],
                                preferred_element_type=jnp.float32) + b3r[...]
  out_ref[...] = 1.0 / (1.0 + jnp.exp(-z))


def _mlp_tc(xw_p, sp, w1a, w1b, b1r, w2, b2r, ww, bwr, w3w, w3d, b3r):
  n = B // BLK
  full = lambda shape: pl.BlockSpec(shape, lambda i: (0, 0))
  return pl.pallas_call(
      _mlp_body,
      grid=(n,),
      in_specs=[
          pl.BlockSpec((BLK, 16), lambda i: (i, 0)),
          pl.BlockSpec((BLK, FIELDS * K), lambda i: (i, 0)),
          full((16, H1)),
          full((FIELDS * K, H1)),
          full((1, H1)),
          full((H1, H2)),
          full((1, H2)),
          full((16, 1)),
          full((1, 1)),
          full((1, 1)),
          full((H2, 1)),
          full((1, 1)),
      ],
      out_specs=pl.BlockSpec((BLK, 1), lambda i: (i, 0)),
      out_shape=jax.ShapeDtypeStruct((B, 1), jnp.float32),
      compiler_params=pltpu.CompilerParams(
          dimension_semantics=("parallel",)),
  )(xw_p, sp, w1a, w1b, b1r, w2, b2r, ww, bwr, w3w, w3d, b3r)


def kernel(x_wide, x_deep, emb, W_wide, b_wide, W1, b1, W2, b2, W3, b3):
  offs = (jnp.arange(FIELDS, dtype=jnp.int32) * VOCAB_PAD)[None, :]
  idx2d = (x_deep.astype(jnp.int32) + offs).reshape(R // CHUNK, CHUNK)
  esel = jnp.stack(
      [jnp.eye(K, 128, k=j * K, dtype=jnp.float32) for j in range(8)])
  embT = jnp.transpose(emb, (0, 2, 1)).reshape(FIELDS * K, VOCAB)
  table = _detile_tc(embT, esel).reshape(FIELDS * VOCAB_PAD, K)
  rows = _gather_sc(idx2d, table)
  sp = rows.reshape(B, FIELDS * K)

  zeros3 = jnp.zeros((3,) + W1.shape[1:], jnp.float32)
  xw_p = jnp.concatenate([x_wide, jnp.zeros((B, 3), jnp.float32)], axis=1)
  w1a = jnp.concatenate([W1[:DENSE], zeros3], axis=0)
  w1b = W1[DENSE:]
  ww = jnp.concatenate([W_wide, jnp.zeros((3, 1), jnp.float32)], axis=0)
  return _mlp_tc(
      xw_p, sp, w1a, w1b, b1.reshape(1, H1), W2, b2.reshape(1, H2),
      ww, b_wide.reshape(1, 1), W3[:1], W3[1:], b3.reshape(1, 1))
